# Initial kernel scaffold; baseline (speedup 1.0000x reference)
import jax
import jax.numpy as jnp
from jax.experimental import pallas as pl


def kernel(node_input, node_attr, edge_src, edge_dst, edge_attr, edge_length_embedded, w_sc, w_lin1, w_lin2, fc_w1, fc_w2):
    raise NotImplementedError("write your pallas kernel here")


# speedup vs baseline: 1.6023x; 1.6023x over previous
import functools
import math

import numpy as np
import jax
import jax.numpy as jnp
from jax import lax
from jax.experimental import pallas as pl
from jax.experimental.pallas import tpu as pltpu
from jax.experimental.pallas import tpu_sc as plsc

N = 10000
E = 160000
D = 128
NB = 8
NH = 64

NCORE = 2
NSUB = 16
CH = 128
EPC = 5120
NCHUNK = EPC // CH
EPAD = NCORE * NSUB * EPC
NPAD = 10240
ROWS_PER_TILE = NPAD // NSUB

_z = np.random.default_rng(0).standard_normal(1000000)
_ACT = float(1.0 / np.sqrt(np.mean((_z / (1.0 + np.exp(-_z))) ** 2)))
C_S = math.sin(math.pi / 8.0)
C_X = math.cos(math.pi / 8.0)


def _w_body(el_ref, ea_ref, w1_ref, w2_ref, o_ref):
    h = jnp.dot(el_ref[...], w1_ref[...], preferred_element_type=jnp.float32)
    h = h * (1.0 / math.sqrt(NB))
    h = h * jax.nn.sigmoid(h) * _ACT
    w = jnp.dot(h, w2_ref[...], preferred_element_type=jnp.float32)
    o_ref[...] = w * ea_ref[...] * (0.25 / math.sqrt(NH))


def _edge_weights():
    BE = 2048
    return pl.pallas_call(
        _w_body,
        grid=(EPAD // BE,),
        in_specs=[
            pl.BlockSpec((BE, NB), lambda i: (i, 0)),
            pl.BlockSpec((BE, 1), lambda i: (i, 0)),
            pl.BlockSpec((NB, NH), lambda i: (0, 0)),
            pl.BlockSpec((NH, D), lambda i: (0, 0)),
        ],
        out_specs=pl.BlockSpec((BE, D), lambda i: (i, 0)),
        out_shape=jax.ShapeDtypeStruct((EPAD, D), jnp.float32),
    )


def _x_body(ni_ref, na_ref, w_ref, o_ref):
    o_ref[...] = jnp.dot(ni_ref[...] * na_ref[...], w_ref[...],
                         preferred_element_type=jnp.float32) * (1.0 / math.sqrt(D))


def _node_lin(ni, na, w):
    BN = 2000
    return pl.pallas_call(
        _x_body,
        grid=(N // BN,),
        in_specs=[
            pl.BlockSpec((BN, D), lambda i: (i, 0)),
            pl.BlockSpec((BN, 1), lambda i: (i, 0)),
            pl.BlockSpec((D, D), lambda i: (0, 0)),
        ],
        out_specs=pl.BlockSpec((BN, D), lambda i: (i, 0)),
        out_shape=jax.ShapeDtypeStruct((N, D), jnp.float32),
    )(ni, na, w)


_mesh = plsc.VectorSubcoreMesh(core_axis_name="c", subcore_axis_name="s")


@functools.partial(
    pl.kernel,
    out_type=jax.ShapeDtypeStruct((NCORE, NPAD, D), jnp.float32),
    mesh=_mesh,
    scratch_types=[
        pltpu.VMEM_SHARED((NPAD, D), jnp.float32),
        pltpu.VMEM((NCHUNK, CH), jnp.int32),
        pltpu.VMEM((NCHUNK, CH), jnp.int32),
        pltpu.VMEM((CH, D), jnp.float32),
        pltpu.VMEM((CH, D), jnp.float32),
        pltpu.SemaphoreType.DMA,
        pltpu.SemaphoreType.DMA,
        pltpu.SemaphoreType.DMA,
    ],
)
def _sc_scatter(x_hbm, src_hbm, dst_hbm, w_hbm, out_hbm,
                agg, src_v, dst_v, wbuf, gbuf, isem, wsem, gsem):
    cid = lax.axis_index("c")
    sid = lax.axis_index("s")

    pltpu.async_copy(src_hbm.at[cid, sid], src_v, isem).wait()
    pltpu.async_copy(dst_hbm.at[cid, sid], dst_v, isem).wait()

    zeros16 = jnp.zeros((16,), jnp.float32)

    def _zb(r, carry):
        for j in range(8):
            wbuf[r, pl.ds(j * 16, 16)] = zeros16
        return carry

    lax.fori_loop(0, CH, _zb, 0)
    for k in range(ROWS_PER_TILE // CH):
        pltpu.sync_copy(wbuf, agg.at[pl.ds(sid * ROWS_PER_TILE + k * CH, CH)])
    plsc.subcore_barrier()

    def _chunk(c, carry):
        pltpu.async_copy(w_hbm.at[cid, sid, pl.ds(c * CH, CH)], wbuf, wsem).wait()
        pltpu.async_copy(x_hbm.at[src_v.at[c]], gbuf, gsem).wait()

        def _mb(r, carry2):
            for j in range(8):
                sl = pl.ds(j * 16, 16)
                gbuf[r, sl] = wbuf[r, sl] * gbuf[r, sl]
            return carry2

        lax.fori_loop(0, CH, _mb, 0)
        pltpu.sync_copy(gbuf, agg.at[dst_v.at[c]], add=True)
        return carry

    lax.fori_loop(0, NCHUNK, _chunk, 0)
    plsc.subcore_barrier()

    pltpu.sync_copy(agg.at[pl.ds(sid * ROWS_PER_TILE, ROWS_PER_TILE)],
                    out_hbm.at[cid, pl.ds(sid * ROWS_PER_TILE, ROWS_PER_TILE)])


def _f_body(ni_ref, na_ref, agg_ref, wsc_ref, wl2_ref, o_ref):
    na = na_ref[...]
    nie = ni_ref[...] * na
    aggs = (agg_ref[0] + agg_ref[1]) * na
    o_ref[...] = (jnp.dot(nie, wsc_ref[...], preferred_element_type=jnp.float32)
                  * (C_S / math.sqrt(D))
                  + jnp.dot(aggs, wl2_ref[...], preferred_element_type=jnp.float32)
                  * (C_X / math.sqrt(D)))


def _final(ni, na, agg2, w_sc, w_lin2):
    BN = 2000
    return pl.pallas_call(
        _f_body,
        grid=(N // BN,),
        in_specs=[
            pl.BlockSpec((BN, D), lambda i: (i, 0)),
            pl.BlockSpec((BN, 1), lambda i: (i, 0)),
            pl.BlockSpec((NCORE, BN, D), lambda i: (0, i, 0)),
            pl.BlockSpec((D, D), lambda i: (0, 0)),
            pl.BlockSpec((D, D), lambda i: (0, 0)),
        ],
        out_specs=pl.BlockSpec((BN, D), lambda i: (i, 0)),
        out_shape=jax.ShapeDtypeStruct((N, D), jnp.float32),
    )(ni, na, agg2, w_sc, w_lin2)


def kernel(node_input, node_attr, edge_src, edge_dst, edge_attr,
           edge_length_embedded, w_sc, w_lin1, w_lin2, fc_w1, fc_w2):
    pad = EPAD - E
    src4 = jnp.reshape(
        jnp.concatenate([edge_src, jnp.zeros((pad,), jnp.int32)]),
        (NCORE, NSUB, NCHUNK, CH))
    dst4 = jnp.reshape(
        jnp.concatenate([edge_dst, jnp.zeros((pad,), jnp.int32)]),
        (NCORE, NSUB, NCHUNK, CH))
    ea_p = jnp.concatenate([edge_attr, jnp.zeros((pad, 1), jnp.float32)])
    el_p = jnp.concatenate(
        [edge_length_embedded, jnp.zeros((pad, NB), jnp.float32)])

    w_edges = _edge_weights()(el_p, ea_p, fc_w1, fc_w2)
    w4 = jnp.reshape(w_edges, (NCORE, NSUB, EPC, D))
    x = _node_lin(node_input, node_attr, w_lin1)
    agg2 = _sc_scatter(x, src4, dst4, w4)
    return _final(node_input, node_attr, agg2[:, :N], w_sc, w_lin2)


# speedup vs baseline: 1.8477x; 1.1531x over previous
import functools
import math

import numpy as np
import jax
import jax.numpy as jnp
from jax import lax
from jax.experimental import pallas as pl
from jax.experimental.pallas import tpu as pltpu
from jax.experimental.pallas import tpu_sc as plsc

N = 10000
E = 160000
D = 128
NB = 8
NH = 64

NCORE = 2
NSUB = 16
CH = 64
EPC = 5120
NCHUNK = EPC // CH
EPAD = NCORE * NSUB * EPC
NPAD = 10240
ROWS_PER_TILE = NPAD // NSUB
NROT = 4

_z = np.random.default_rng(0).standard_normal(1000000)
_ACT = float(1.0 / np.sqrt(np.mean((_z / (1.0 + np.exp(-_z))) ** 2)))
C_S = math.sin(math.pi / 8.0)
C_X = math.cos(math.pi / 8.0)


def _w_body(el_ref, ea_ref, w1_ref, w2_ref, o_ref):
    h = jnp.dot(el_ref[...], w1_ref[...], preferred_element_type=jnp.float32)
    h = h * (1.0 / math.sqrt(NB))
    h = h * jax.nn.sigmoid(h) * _ACT
    w = jnp.dot(h, w2_ref[...], preferred_element_type=jnp.float32)
    o_ref[...] = w * ea_ref[...] * (0.25 / math.sqrt(NH))


def _edge_weights():
    BE = 2048
    return pl.pallas_call(
        _w_body,
        grid=(EPAD // BE,),
        in_specs=[
            pl.BlockSpec((BE, NB), lambda i: (i, 0)),
            pl.BlockSpec((BE, 1), lambda i: (i, 0)),
            pl.BlockSpec((NB, NH), lambda i: (0, 0)),
            pl.BlockSpec((NH, D), lambda i: (0, 0)),
        ],
        out_specs=pl.BlockSpec((BE, D), lambda i: (i, 0)),
        out_shape=jax.ShapeDtypeStruct((EPAD, D), jnp.float32),
    )


def _x_body(ni_ref, na_ref, w_ref, o_ref):
    o_ref[...] = jnp.dot(ni_ref[...] * na_ref[...], w_ref[...],
                         preferred_element_type=jnp.float32) * (1.0 / math.sqrt(D))


def _node_lin(ni, na, w):
    BN = 2000
    return pl.pallas_call(
        _x_body,
        grid=(N // BN,),
        in_specs=[
            pl.BlockSpec((BN, D), lambda i: (i, 0)),
            pl.BlockSpec((BN, 1), lambda i: (i, 0)),
            pl.BlockSpec((D, D), lambda i: (0, 0)),
        ],
        out_specs=pl.BlockSpec((BN, D), lambda i: (i, 0)),
        out_shape=jax.ShapeDtypeStruct((N, D), jnp.float32),
    )(ni, na, w)


_mesh = plsc.VectorSubcoreMesh(core_axis_name="c", subcore_axis_name="s")


@functools.partial(
    pl.kernel,
    out_type=jax.ShapeDtypeStruct((NCORE, NPAD, D), jnp.float32),
    mesh=_mesh,
    scratch_types=[
        pltpu.VMEM_SHARED((NPAD, D), jnp.float32),
        pltpu.VMEM((NROT, CH), jnp.int32),
        pltpu.VMEM((NROT, CH), jnp.int32),
        pltpu.VMEM((CH, D), jnp.float32),
        pltpu.VMEM((NROT, CH, D), jnp.float32),
        pltpu.SemaphoreType.DMA,
        pltpu.SemaphoreType.DMA,
        pltpu.SemaphoreType.DMA,
        pltpu.SemaphoreType.DMA,
        pltpu.SemaphoreType.DMA,
    ],
)
def _sc_scatter(x_hbm, src_hbm, dst_hbm, w_hbm, out_hbm,
                agg, src_v, dst_i, wbuf, gbuf, isem, dsem, wsem, gsem, ssem):
    cid = lax.axis_index("c")
    sid = lax.axis_index("s")

    for b in range(NROT):
        pltpu.async_copy(src_hbm.at[cid, sid, b], src_v.at[b], isem).wait()
        pltpu.async_copy(dst_hbm.at[cid, sid, b], dst_i.at[b], dsem).wait()

    pltpu.async_copy(w_hbm.at[cid, sid, pl.ds(0, CH)], wbuf, wsem)
    pltpu.async_copy(x_hbm.at[src_v.at[0]], gbuf.at[0], gsem)
    pltpu.async_copy(x_hbm.at[src_v.at[1]], gbuf.at[1], gsem)

    zeros16 = jnp.zeros((16,), jnp.float32)

    @plsc.parallel_loop(0, CH)
    def _zb(r):
        for j in range(8):
            gbuf[NROT - 1, r, pl.ds(j * 16, 16)] = zeros16

    for k in range(ROWS_PER_TILE // CH):
        pltpu.sync_copy(gbuf.at[NROT - 1],
                        agg.at[pl.ds(sid * ROWS_PER_TILE + k * CH, CH)])
    plsc.subcore_barrier()

    def _outer(i4, carry):
        for b in range(NROT):
            c = i4 * NROT + b
            b2 = (b + 2) % NROT
            pltpu.make_async_copy(
                w_hbm.at[cid, sid, pl.ds(c * CH, CH)], wbuf, wsem).wait()
            pltpu.make_async_copy(
                x_hbm.at[src_v.at[b]], gbuf.at[b], gsem).wait()

            @pl.when(c + NROT < NCHUNK)
            def _():
                pltpu.async_copy(
                    src_hbm.at[cid, sid, c + NROT], src_v.at[b], isem)

            @plsc.parallel_loop(0, CH)
            def _mb(r):
                for j in range(8):
                    sl = pl.ds(j * 16, 16)
                    gbuf[b, r, sl] = wbuf[r, sl] * gbuf[b, r, sl]

            @pl.when(c + 1 < NCHUNK)
            def _():
                pltpu.async_copy(
                    w_hbm.at[cid, sid, pl.ds((c + 1) * CH, CH)], wbuf, wsem)

            @pl.when(c >= 2)
            def _():
                pltpu.make_async_copy(
                    gbuf.at[b2], agg.at[dst_i.at[b2]], ssem).wait()

            @pl.when((c >= 2) & (c + 2 < NCHUNK))
            def _():
                pltpu.async_copy(
                    dst_hbm.at[cid, sid, c + 2], dst_i.at[b2], dsem)

            @pl.when(c + 2 < NCHUNK)
            def _():
                @pl.when(c >= 2)
                def _():
                    pltpu.make_async_copy(
                        src_hbm.at[cid, sid, c + 2], src_v.at[b2], isem).wait()

                pltpu.async_copy(x_hbm.at[src_v.at[b2]], gbuf.at[b2], gsem)

            @pl.when(c >= NROT)
            def _():
                pltpu.make_async_copy(
                    dst_hbm.at[cid, sid, c], dst_i.at[b], dsem).wait()

            pltpu.async_copy(gbuf.at[b], agg.at[dst_i.at[b]], ssem, add=True)
        return carry

    lax.fori_loop(0, NCHUNK // NROT, _outer, 0)

    pltpu.make_async_copy(
        gbuf.at[(NCHUNK - 2) % NROT],
        agg.at[dst_i.at[(NCHUNK - 2) % NROT]], ssem).wait()
    pltpu.make_async_copy(
        gbuf.at[(NCHUNK - 1) % NROT],
        agg.at[dst_i.at[(NCHUNK - 1) % NROT]], ssem).wait()
    plsc.subcore_barrier()

    pltpu.sync_copy(agg.at[pl.ds(sid * ROWS_PER_TILE, ROWS_PER_TILE)],
                    out_hbm.at[cid, pl.ds(sid * ROWS_PER_TILE, ROWS_PER_TILE)])


def _f_body(ni_ref, na_ref, agg_ref, wsc_ref, wl2_ref, o_ref):
    na = na_ref[...]
    nie = ni_ref[...] * na
    aggs = (agg_ref[0] + agg_ref[1]) * na
    o_ref[...] = (jnp.dot(nie, wsc_ref[...], preferred_element_type=jnp.float32)
                  * (C_S / math.sqrt(D))
                  + jnp.dot(aggs, wl2_ref[...], preferred_element_type=jnp.float32)
                  * (C_X / math.sqrt(D)))


def _final(ni, na, agg2, w_sc, w_lin2):
    BN = 2000
    return pl.pallas_call(
        _f_body,
        grid=(N // BN,),
        in_specs=[
            pl.BlockSpec((BN, D), lambda i: (i, 0)),
            pl.BlockSpec((BN, 1), lambda i: (i, 0)),
            pl.BlockSpec((NCORE, BN, D), lambda i: (0, i, 0)),
            pl.BlockSpec((D, D), lambda i: (0, 0)),
            pl.BlockSpec((D, D), lambda i: (0, 0)),
        ],
        out_specs=pl.BlockSpec((BN, D), lambda i: (i, 0)),
        out_shape=jax.ShapeDtypeStruct((N, D), jnp.float32),
    )(ni, na, agg2, w_sc, w_lin2)


def kernel(node_input, node_attr, edge_src, edge_dst, edge_attr,
           edge_length_embedded, w_sc, w_lin1, w_lin2, fc_w1, fc_w2):
    pad = EPAD - E
    src4 = jnp.reshape(
        jnp.concatenate([edge_src, jnp.zeros((pad,), jnp.int32)]),
        (NCORE, NSUB, NCHUNK, CH))
    dst4 = jnp.reshape(
        jnp.concatenate([edge_dst, jnp.zeros((pad,), jnp.int32)]),
        (NCORE, NSUB, NCHUNK, CH))
    ea_p = jnp.concatenate([edge_attr, jnp.zeros((pad, 1), jnp.float32)])
    el_p = jnp.concatenate(
        [edge_length_embedded, jnp.zeros((pad, NB), jnp.float32)])

    w_edges = _edge_weights()(el_p, ea_p, fc_w1, fc_w2)
    w4 = jnp.reshape(w_edges, (NCORE, NSUB, EPC, D))
    x = _node_lin(node_input, node_attr, w_lin1)
    agg2 = _sc_scatter(x, src4, dst4, w4)
    return _final(node_input, node_attr, agg2[:, :N], w_sc, w_lin2)
